# baseline (speedup 1.0000x reference)
import jax
import jax.numpy as jnp
from jax.experimental import pallas as pl

N = 10000
D = 128
H1 = 64
H2 = 32

_BLK = 2000
_HALF = _BLK // 2


def _perm64():
    j = jax.lax.broadcasted_iota(jnp.int32, (64, 64), 0)
    k = jax.lax.broadcasted_iota(jnp.int32, (64, 64), 1)
    tgt = jnp.where(j % 2 == 0, j // 2, 32 + (j - 1) // 2)
    return (k == tgt).astype(jnp.float32)


def _perm32():
    j = jax.lax.broadcasted_iota(jnp.int32, (32, 32), 0)
    k = jax.lax.broadcasted_iota(jnp.int32, (32, 32), 1)
    tgt = 8 * (j % 4) + j // 4
    return (k == tgt).astype(jnp.float32)


def _fused_kernel(x_ref, buf_ref, out_ref):
    bf16 = jnp.bfloat16
    f32 = jnp.float32
    x = x_ref[...].astype(bf16)

    a1 = buf_ref[0:128, :] + buf_ref[192:320, :]
    b1 = buf_ref[384:512, :] + buf_ref[576:704, :]
    w1 = jnp.concatenate([a1 * 0.5, b1], axis=1).astype(bf16)

    bias = buf_ref[968:976, :]
    bias1 = jnp.concatenate([bias[0:1, :] * 0.5, bias[1:2, :]], axis=1)

    pe64 = _perm64()
    sz = buf_ref[768:800, :] + buf_ref[816:848, :]
    a2 = jnp.dot(pe64, jnp.concatenate([sz[:, :H2], sz[:, H2:]], axis=0),
                 preferred_element_type=f32)
    sh = buf_ref[864:896, :] + buf_ref[912:944, :]
    b2 = jnp.dot(pe64, jnp.concatenate([sh[:, :H2], sh[:, H2:]], axis=0),
                 preferred_element_type=f32)
    zz = jnp.zeros((H1, H2), dtype=f32)
    w2 = jnp.concatenate([
        jnp.concatenate([a2 * 0.25, zz, b2 * 0.5, zz], axis=1),
        jnp.concatenate([zz, a2 * 0.25, zz, b2 * 0.5], axis=1),
    ], axis=0).astype(bf16)
    bz2 = bias[2:3, :H2]
    bh2 = bias[2:3, H2:]
    bias2 = jnp.concatenate([bz2 * 0.5, bz2 * 0.5, bh2, bh2], axis=1)

    sl = buf_ref[960:968, :]
    wl1g = jnp.concatenate([sl[:, 0:16], sl[:, 16:32],
                            sl[:, 32:48], sl[:, 48:64]], axis=0)
    wl1 = jnp.dot(_perm32(), wl1g, preferred_element_type=f32) * 0.5
    z2 = jnp.zeros((H2, 16), dtype=f32)
    w3 = jnp.concatenate([
        jnp.concatenate([wl1, z2], axis=1),
        jnp.concatenate([z2, wl1], axis=1),
    ], axis=0).astype(bf16)
    bl1 = bias[3:4, 16:32]
    bias3 = jnp.concatenate([bl1, bl1], axis=1)

    wl2 = bias[3:4, 0:16]
    wl2b = jnp.concatenate([wl2, wl2], axis=1)
    bl2 = bias[3:4, 32:33]

    t1 = jnp.tanh(jnp.dot(x, w1, preferred_element_type=f32) + bias1)
    g1 = jax.nn.relu((1.0 - t1[:, :H1]) * t1[:, H1:])
    g1p = jnp.concatenate([g1[:_HALF], g1[_HALF:]], axis=1).astype(bf16)

    t2 = jnp.tanh(jnp.dot(g1p, w2, preferred_element_type=f32) + bias2)
    g2 = jax.nn.relu((1.0 - t2[:, :H1]) * t2[:, H1:])

    h3 = jax.nn.relu(jnp.dot(g2.astype(bf16), w3,
                             preferred_element_type=f32) + bias3)
    prod = h3 * wl2b
    y1 = jnp.sum(prod[:, :16], axis=1, keepdims=True)
    y2 = jnp.sum(prod[:, 16:], axis=1, keepdims=True)
    out_ref[:_HALF, :] = y1 + bl2
    out_ref[_HALF:, :] = y2 + bl2


def kernel(x, edge_index, edge_weight,
           W_z1, b_z1, W_r1, b_r1, W_h1, b_h1,
           W_z2, b_z2, W_r2, b_r2, W_h2, b_h2,
           W_l1, b_l1, W_l2, b_l2):
    del edge_index, edge_weight, W_r1, b_r1, W_r2, b_r2

    buf = jnp.concatenate([
        W_z1.ravel(), W_h1.ravel(), W_z2.ravel(), W_h2.ravel(),
        W_l1.ravel(), b_z1, b_h1, b_z2, b_h2,
        W_l2.ravel(), b_l1, b_l2,
        jnp.zeros((287,), jnp.float32),
    ]).reshape(976, 64)

    out = pl.pallas_call(
        _fused_kernel,
        grid=(N // _BLK,),
        in_specs=[
            pl.BlockSpec((_BLK, D), lambda i: (i, 0)),
            pl.BlockSpec((976, 64), lambda i: (0, 0)),
        ],
        out_specs=pl.BlockSpec((_BLK, 1), lambda i: (i, 0)),
        out_shape=jax.ShapeDtypeStruct((N, 1), jnp.float32),
    )(x, buf)
    return out


# speedup vs baseline: 1.0003x; 1.0003x over previous
import jax
import jax.numpy as jnp
from jax.experimental import pallas as pl
from jax.experimental.pallas import tpu as pltpu

N = 10000
D = 128
H1 = 64
H2 = 32

_BLK = 2000
_HALF = _BLK // 2


def _perm64():
    j = jax.lax.broadcasted_iota(jnp.int32, (64, 64), 0)
    k = jax.lax.broadcasted_iota(jnp.int32, (64, 64), 1)
    tgt = jnp.where(j % 2 == 0, j // 2, 32 + (j - 1) // 2)
    return (k == tgt).astype(jnp.float32)


def _perm32():
    j = jax.lax.broadcasted_iota(jnp.int32, (32, 32), 0)
    k = jax.lax.broadcasted_iota(jnp.int32, (32, 32), 1)
    tgt = 8 * (j % 4) + j // 4
    return (k == tgt).astype(jnp.float32)


def _fused_kernel(x_ref, buf_ref, out_ref, w1s, w2s, w3s, bs):
    bf16 = jnp.bfloat16
    f32 = jnp.float32

    @pl.when(pl.program_id(0) == 0)
    def _prep():
        a1 = buf_ref[0:128, :] + buf_ref[192:320, :]
        b1 = buf_ref[384:512, :] + buf_ref[576:704, :]
        w1s[...] = jnp.concatenate([a1 * 0.5, b1], axis=1).astype(bf16)

        bias = buf_ref[968:976, :]
        bs[0:1, :] = jnp.concatenate([bias[0:1, :] * 0.5, bias[1:2, :]],
                                     axis=1)

        pe64 = _perm64()
        sz = buf_ref[768:800, :] + buf_ref[816:848, :]
        a2 = jnp.dot(pe64, jnp.concatenate([sz[:, :H2], sz[:, H2:]], axis=0),
                     preferred_element_type=f32)
        sh = buf_ref[864:896, :] + buf_ref[912:944, :]
        b2 = jnp.dot(pe64, jnp.concatenate([sh[:, :H2], sh[:, H2:]], axis=0),
                     preferred_element_type=f32)
        zz = jnp.zeros((H1, H2), dtype=f32)
        w2s[...] = jnp.concatenate([
            jnp.concatenate([a2 * 0.25, zz, b2 * 0.5, zz], axis=1),
            jnp.concatenate([zz, a2 * 0.25, zz, b2 * 0.5], axis=1),
        ], axis=0).astype(bf16)
        bz2 = bias[2:3, :H2]
        bh2 = bias[2:3, H2:]
        bs[1:2, :] = jnp.concatenate([bz2 * 0.5, bz2 * 0.5, bh2, bh2], axis=1)

        sl = buf_ref[960:968, :]
        wl1g = jnp.concatenate([sl[:, 0:16], sl[:, 16:32],
                                sl[:, 32:48], sl[:, 48:64]], axis=0)
        wl1 = jnp.dot(_perm32(), wl1g, preferred_element_type=f32) * 0.5
        z2 = jnp.zeros((H2, 16), dtype=f32)
        w3s[...] = jnp.concatenate([
            jnp.concatenate([wl1, z2], axis=1),
            jnp.concatenate([z2, wl1], axis=1),
        ], axis=0).astype(bf16)
        bl1 = bias[3:4, 16:32]
        wl2 = bias[3:4, 0:16]
        bl2 = bias[3:4, 32:33]
        row = jnp.concatenate(
            [bl1, bl1, wl2, wl2, bl2, jnp.zeros((1, 63), f32)], axis=1)
        bs[2:3, :] = row

    x = x_ref[...].astype(bf16)
    t1 = jnp.tanh(jnp.dot(x, w1s[...], preferred_element_type=f32)
                  + bs[0:1, :])
    g1 = jax.nn.relu((1.0 - t1[:, :H1]) * t1[:, H1:])
    g1p = jnp.concatenate([g1[:_HALF], g1[_HALF:]], axis=1).astype(bf16)

    t2 = jnp.tanh(jnp.dot(g1p, w2s[...], preferred_element_type=f32)
                  + bs[1:2, :])
    g2 = jax.nn.relu((1.0 - t2[:, :H1]) * t2[:, H1:])

    h3 = jax.nn.relu(jnp.dot(g2.astype(bf16), w3s[...],
                             preferred_element_type=f32) + bs[2:3, :H2])
    prod = h3 * bs[2:3, H2:H1]
    y1 = jnp.sum(prod[:, :16], axis=1, keepdims=True)
    y2 = jnp.sum(prod[:, 16:], axis=1, keepdims=True)
    bl2 = bs[2:3, H1:H1 + 1]
    out_ref[:_HALF, :] = y1 + bl2
    out_ref[_HALF:, :] = y2 + bl2


def kernel(x, edge_index, edge_weight,
           W_z1, b_z1, W_r1, b_r1, W_h1, b_h1,
           W_z2, b_z2, W_r2, b_r2, W_h2, b_h2,
           W_l1, b_l1, W_l2, b_l2):
    del edge_index, edge_weight, W_r1, b_r1, W_r2, b_r2

    buf = jnp.concatenate([
        W_z1.ravel(), W_h1.ravel(), W_z2.ravel(), W_h2.ravel(),
        W_l1.ravel(), b_z1, b_h1, b_z2, b_h2,
        W_l2.ravel(), b_l1, b_l2,
        jnp.zeros((287,), jnp.float32),
    ]).reshape(976, 64)

    out = pl.pallas_call(
        _fused_kernel,
        grid=(N // _BLK,),
        in_specs=[
            pl.BlockSpec((_BLK, D), lambda i: (i, 0)),
            pl.BlockSpec((976, 64), lambda i: (0, 0)),
        ],
        out_specs=pl.BlockSpec((_BLK, 1), lambda i: (i, 0)),
        out_shape=jax.ShapeDtypeStruct((N, 1), jnp.float32),
        scratch_shapes=[
            pltpu.VMEM((128, 128), jnp.bfloat16),
            pltpu.VMEM((128, 128), jnp.bfloat16),
            pltpu.VMEM((64, 32), jnp.bfloat16),
            pltpu.VMEM((8, 128), jnp.float32),
        ],
    )(x, buf)
    return out


# speedup vs baseline: 1.4683x; 1.4679x over previous
import jax
import jax.numpy as jnp
from jax.experimental import pallas as pl
from jax.experimental.pallas import tpu as pltpu

N = 10000
D = 128
H1 = 64
H2 = 32

_BLK = 2000
_HALF = _BLK // 2


def _fused_kernel(xa_ref, wa_ref, wb_ref, wl1_ref, wl2_ref, bv_ref,
                  out_ref, w1s, w2s, w3s, w4s, bs):
    bf16 = jnp.bfloat16
    f32 = jnp.float32

    @pl.when(pl.program_id(0) == 0)
    def _prep():
        a1 = wa_ref[0:128, :] + wa_ref[192:320, :]
        b1 = wa_ref[384:512, :] + wa_ref[576:704, :]
        w1s[...] = jnp.concatenate([a1 * 0.5, b1], axis=1).astype(bf16)
        bz1 = bv_ref[0:1, 0:64]
        bh1 = bv_ref[0:1, 64:128]
        bs[0:1, :] = jnp.concatenate([bz1 * 0.5, bh1], axis=1)

        a2 = wb_ref[0:64, :] + wb_ref[96:160, :]
        b2 = wb_ref[192:256, :] + wb_ref[288:352, :]
        zz = jnp.zeros((H1, H2), dtype=f32)
        w2s[...] = jnp.concatenate([
            jnp.concatenate([a2 * 0.25, zz, b2 * 0.5, zz], axis=1),
            jnp.concatenate([zz, a2 * 0.25, zz, b2 * 0.5], axis=1),
        ], axis=0).astype(bf16)
        bz2 = bv_ref[0:1, 128:160]
        bh2 = bv_ref[0:1, 160:192]
        bs[1:2, :] = jnp.concatenate([bz2 * 0.5, bz2 * 0.5, bh2, bh2], axis=1)

        wl1 = wl1_ref[...] * 0.5
        z2 = jnp.zeros((H2, 16), dtype=f32)
        w3s[...] = jnp.concatenate([
            jnp.concatenate([wl1, z2], axis=1),
            jnp.concatenate([z2, wl1], axis=1),
        ], axis=0).astype(bf16)
        bl1 = bv_ref[0:1, 192:208]
        bl2 = bv_ref[0:1, 208:209]
        bs[2:3, :] = jnp.concatenate(
            [bl1, bl1, bl2, jnp.zeros((1, 95), f32)], axis=1)

        z3 = jnp.zeros((16, 1), dtype=f32)
        w4s[...] = jnp.concatenate([
            jnp.concatenate([wl2_ref[...], z3], axis=1),
            jnp.concatenate([z3, wl2_ref[...]], axis=1),
        ], axis=0).astype(bf16)

    x = xa_ref[...].astype(bf16)
    t1 = jnp.tanh(jnp.dot(x, w1s[...], preferred_element_type=f32)
                  + bs[0:1, :])
    g1 = jax.nn.relu((1.0 - t1[:, :H1]) * t1[:, H1:])
    g1p = jnp.concatenate([g1[:_HALF], g1[_HALF:]], axis=1).astype(bf16)

    t2 = jnp.tanh(jnp.dot(g1p, w2s[...], preferred_element_type=f32)
                  + bs[1:2, :])
    g2 = jax.nn.relu((1.0 - t2[:, :H1]) * t2[:, H1:])

    h3 = jax.nn.relu(jnp.dot(g2.astype(bf16), w3s[...],
                             preferred_element_type=f32) + bs[2:3, :H2])
    y = (jnp.dot(h3.astype(bf16), w4s[...], preferred_element_type=f32)
         + bs[2:3, H2:H2 + 1])
    out_ref[:_HALF, :] = y[:, 0:1]
    out_ref[_HALF:, :] = y[:, 1:2]


def kernel(x, edge_index, edge_weight,
           W_z1, b_z1, W_r1, b_r1, W_h1, b_h1,
           W_z2, b_z2, W_r2, b_r2, W_h2, b_h2,
           W_l1, b_l1, W_l2, b_l2):
    del edge_index, edge_weight, W_r1, b_r1, W_r2, b_r2

    bufa = jnp.concatenate([W_z1.reshape(384, 64), W_h1.reshape(384, 64)],
                           axis=0)
    bufb = jnp.concatenate([W_z2.reshape(192, 32), W_h2.reshape(192, 32)],
                           axis=0)
    bvec = jnp.concatenate([b_z1, b_h1, b_z2, b_h2, b_l1, b_l2]
                           ).reshape(1, 209)

    out = pl.pallas_call(
        _fused_kernel,
        grid=(N // _BLK,),
        in_specs=[
            pl.BlockSpec((_BLK, D), lambda i: (i, 0)),
            pl.BlockSpec((768, 64), lambda i: (0, 0)),
            pl.BlockSpec((384, 32), lambda i: (0, 0)),
            pl.BlockSpec((32, 16), lambda i: (0, 0)),
            pl.BlockSpec((16, 1), lambda i: (0, 0)),
            pl.BlockSpec((1, 209), lambda i: (0, 0)),
        ],
        out_specs=pl.BlockSpec((_BLK, 1), lambda i: (i, 0)),
        out_shape=jax.ShapeDtypeStruct((N, 1), jnp.float32),
        scratch_shapes=[
            pltpu.VMEM((128, 128), jnp.bfloat16),
            pltpu.VMEM((128, 128), jnp.bfloat16),
            pltpu.VMEM((64, 32), jnp.bfloat16),
            pltpu.VMEM((32, 2), jnp.bfloat16),
            pltpu.VMEM((8, 128), jnp.float32),
        ],
    )(x, bufa, bufb, W_l1, W_l2, bvec)
    return out


# speedup vs baseline: 1.4959x; 1.0187x over previous
import jax
import jax.numpy as jnp
from jax.experimental import pallas as pl
from jax.experimental.pallas import tpu as pltpu

N = 10000
D = 128
H1 = 64
H2 = 32

_BLK = 2000
_HALF = _BLK // 2


def _fused_kernel(xa_ref,
                  wz1_ref, bz1_ref, wh1_ref, bh1_ref,
                  wz2_ref, bz2_ref, wh2_ref, bh2_ref,
                  wl1_ref, bl1_ref, wl2_ref, bl2_ref,
                  out_ref, w1s, w2s, w3s, w4s, bs):
    bf16 = jnp.bfloat16
    f32 = jnp.float32

    @pl.when(pl.program_id(0) == 0)
    def _prep():
        a1 = wz1_ref[0, :D, :] + wz1_ref[1, :D, :]
        b1 = wh1_ref[0, :D, :] + wh1_ref[1, :D, :]
        w1s[...] = jnp.concatenate([a1 * 0.5, b1], axis=1).astype(bf16)
        bs[0:1, :] = jnp.concatenate([bz1_ref[...] * 0.5, bh1_ref[...]],
                                     axis=1)

        a2 = wz2_ref[0, :H1, :] + wz2_ref[1, :H1, :]
        b2 = wh2_ref[0, :H1, :] + wh2_ref[1, :H1, :]
        zz = jnp.zeros((H1, H2), dtype=f32)
        w2s[...] = jnp.concatenate([
            jnp.concatenate([a2 * 0.25, zz, b2 * 0.5, zz], axis=1),
            jnp.concatenate([zz, a2 * 0.25, zz, b2 * 0.5], axis=1),
        ], axis=0).astype(bf16)
        bz2 = bz2_ref[...] * 0.5
        bs[1:2, :] = jnp.concatenate([bz2, bz2, bh2_ref[...], bh2_ref[...]],
                                     axis=1)

        wl1 = wl1_ref[...] * 0.5
        z2 = jnp.zeros((H2, 16), dtype=f32)
        w3s[...] = jnp.concatenate([
            jnp.concatenate([wl1, z2], axis=1),
            jnp.concatenate([z2, wl1], axis=1),
        ], axis=0).astype(bf16)
        bl1 = bl1_ref[...]
        bs[2:3, :] = jnp.concatenate(
            [bl1, bl1, bl2_ref[...], jnp.zeros((1, 95), f32)], axis=1)

        z3 = jnp.zeros((16, 1), dtype=f32)
        w4s[...] = jnp.concatenate([
            jnp.concatenate([wl2_ref[...], z3], axis=1),
            jnp.concatenate([z3, wl2_ref[...]], axis=1),
        ], axis=0).astype(bf16)

    x = xa_ref[...].astype(bf16)
    t1 = jnp.tanh(jnp.dot(x, w1s[...], preferred_element_type=f32)
                  + bs[0:1, :])
    g1 = jax.nn.relu((1.0 - t1[:, :H1]) * t1[:, H1:])
    g1p = jnp.concatenate([g1[:_HALF], g1[_HALF:]], axis=1).astype(bf16)

    t2 = jnp.tanh(jnp.dot(g1p, w2s[...], preferred_element_type=f32)
                  + bs[1:2, :])
    g2 = jax.nn.relu((1.0 - t2[:, :H1]) * t2[:, H1:])

    h3 = jax.nn.relu(jnp.dot(g2.astype(bf16), w3s[...],
                             preferred_element_type=f32) + bs[2:3, :H2])
    y = (jnp.dot(h3.astype(bf16), w4s[...], preferred_element_type=f32)
         + bs[2:3, H2:H2 + 1])
    out_ref[:_HALF, :] = y[:, 0:1]
    out_ref[_HALF:, :] = y[:, 1:2]


def kernel(x, edge_index, edge_weight,
           W_z1, b_z1, W_r1, b_r1, W_h1, b_h1,
           W_z2, b_z2, W_r2, b_r2, W_h2, b_h2,
           W_l1, b_l1, W_l2, b_l2):
    del edge_index, edge_weight, W_r1, b_r1, W_r2, b_r2

    def wspec(a):
        shp = a.shape
        return pl.BlockSpec(shp, lambda i: (0,) * len(shp))

    biases = [b.reshape(1, -1) for b in (b_z1, b_h1, b_z2, b_h2, b_l1, b_l2)]
    bz1, bh1, bz2, bh2, bl1, bl2 = biases

    out = pl.pallas_call(
        _fused_kernel,
        grid=(N // _BLK,),
        in_specs=[
            pl.BlockSpec((_BLK, D), lambda i: (i, 0)),
            wspec(W_z1), wspec(bz1), wspec(W_h1), wspec(bh1),
            wspec(W_z2), wspec(bz2), wspec(W_h2), wspec(bh2),
            wspec(W_l1), wspec(bl1), wspec(W_l2), wspec(bl2),
        ],
        out_specs=pl.BlockSpec((_BLK, 1), lambda i: (i, 0)),
        out_shape=jax.ShapeDtypeStruct((N, 1), jnp.float32),
        scratch_shapes=[
            pltpu.VMEM((128, 128), jnp.bfloat16),
            pltpu.VMEM((128, 128), jnp.bfloat16),
            pltpu.VMEM((64, 32), jnp.bfloat16),
            pltpu.VMEM((32, 2), jnp.bfloat16),
            pltpu.VMEM((8, 128), jnp.float32),
        ],
    )(x, W_z1, bz1, W_h1, bh1, W_z2, bz2, W_h2, bh2, W_l1, bl1, W_l2, bl2)
    return out
